# baseline (speedup 1.0000x reference)
import functools

import jax
import jax.numpy as jnp
from jax import lax
from jax.experimental import pallas as pl
from jax.experimental.pallas import tpu as pltpu
from jax.experimental.pallas import tpu_sc as plsc

VOCAB = 1000000
DIM = 64
B = 16384
L = 20
W = 128

NC = 2
NS = 16
NW = NC * NS

NTOT = B * L
PER_W = NTOT // NW
SUB = 128
NSUB = 2
CH = SUB * NSUB
NCHUNK = PER_W // CH

VCH = 128
NVCH = VOCAB // VCH
VTAIL = VOCAB - NVCH * VCH
VTAIL0 = NVCH * VCH
VPAD = VTAIL0 + VCH
VCH_W = -(-NVCH // NW)


def _tr_kernel(tin_hbm, tout_hbm, tailin_hbm, tailout_hbm, lin_hbm, lout_hbm,
               dbuf_c, dbuf_x, lines_c, lines_x, sem):
    wid = lax.axis_index("s") * NC + lax.axis_index("c")

    iota16 = lax.iota(jnp.int32, 16)
    kconst = (iota16 >= 8).astype(jnp.int32)
    r8 = iota16 & 7

    def chunk_body(i, _):
        k = wid + i * NW

        @pl.when(k < NVCH)
        def _():
            v0 = pl.multiple_of(k * VCH, VCH)
            copies = []
            for dj in range(8):
                copies.append(pltpu.async_copy(
                    tin_hbm.at[pl.ds(dj * 8, 8), pl.ds(v0, VCH)],
                    dbuf_c.at[dj], sem))
                copies.append(pltpu.async_copy(
                    tout_hbm.at[pl.ds(dj * 8, 8), pl.ds(v0, VCH)],
                    dbuf_x.at[dj], sem))
            for cp in copies:
                cp.wait()

            def q_body(q, _):
                rowi = q * 2 + kconst
                coli = q * 2 + kconst
                for dj in range(8):
                    cidx = dj * 8 + r8
                    djv = jnp.full((16,), dj, jnp.int32)
                    vc = plsc.load_gather(dbuf_c, [djv, r8, coli])
                    plsc.store_scatter(lines_c, [rowi, cidx], vc)
                    vx = plsc.load_gather(dbuf_x, [djv, r8, coli])
                    plsc.store_scatter(lines_x, [rowi, cidx], vx)
                return 0

            lax.fori_loop(0, VCH // 2, q_body, 0)

            pltpu.sync_copy(lines_c, lin_hbm.at[pl.ds(v0, VCH)])
            pltpu.sync_copy(lines_x, lout_hbm.at[pl.ds(v0, VCH)])
        return 0

    lax.fori_loop(0, VCH_W, chunk_body, 0)

    @pl.when(wid == NVCH % NW)
    def _():
        copies = []
        for dj in range(8):
            copies.append(pltpu.async_copy(
                tailin_hbm.at[pl.ds(dj * 8, 8)], dbuf_c.at[dj], sem))
            copies.append(pltpu.async_copy(
                tailout_hbm.at[pl.ds(dj * 8, 8)], dbuf_x.at[dj], sem))
        for cp in copies:
            cp.wait()

        def q_body(q, _):
            rowi = q * 2 + kconst
            coli = q * 2 + kconst
            for dj in range(8):
                cidx = dj * 8 + r8
                djv = jnp.full((16,), dj, jnp.int32)
                vc = plsc.load_gather(dbuf_c, [djv, r8, coli])
                plsc.store_scatter(lines_c, [rowi, cidx], vc)
                vx = plsc.load_gather(dbuf_x, [djv, r8, coli])
                plsc.store_scatter(lines_x, [rowi, cidx], vx)
            return 0

        lax.fori_loop(0, VCH // 2, q_body, 0)

        pltpu.sync_copy(lines_c, lin_hbm.at[pl.ds(VTAIL0, VCH)])
        pltpu.sync_copy(lines_x, lout_hbm.at[pl.ds(VTAIL0, VCH)])


def _sc_kernel(cw_hbm, xw_hbm, in_hbm, out_emb_hbm, out_hbm,
               idx_c, idx_x, crows, xrows, scores, sem):
    wid = lax.axis_index("s") * NC + lax.axis_index("c")

    def chunk_body(c, _):
        pltpu.sync_copy(cw_hbm.at[wid, c], idx_c)
        pltpu.sync_copy(xw_hbm.at[wid, c], idx_x)

        copies = []
        for j in range(NSUB):
            copies.append(
                pltpu.async_copy(in_hbm.at[idx_c.at[j]], crows.at[j], sem))
            copies.append(
                pltpu.async_copy(out_emb_hbm.at[idx_x.at[j]], xrows.at[j], sem))
        for cp in copies:
            cp.wait()

        iota16 = lax.iota(jnp.int32, 16)
        for j in range(NSUB):
            def grp_body(g, _):
                out16 = jnp.zeros((16,), jnp.float32)
                for p in range(16):
                    i = g * 16 + p
                    acc = (crows[j, i, pl.ds(0, 16)] * xrows[j, i, pl.ds(0, 16)]
                           + crows[j, i, pl.ds(16, 16)] * xrows[j, i, pl.ds(16, 16)])
                    acc = acc + crows[j, i, pl.ds(32, 16)] * xrows[j, i, pl.ds(32, 16)]
                    acc = acc + crows[j, i, pl.ds(48, 16)] * xrows[j, i, pl.ds(48, 16)]
                    s = jnp.sum(acc)
                    out16 = jnp.where(iota16 == p, lax.broadcast(s, (16,)), out16)
                scores[j, pl.ds(g * 16, 16)] = out16
                return 0
            lax.fori_loop(0, SUB // 16, grp_body, 0)

        pltpu.sync_copy(scores, out_hbm.at[wid, c])
        return 0

    lax.fori_loop(0, NCHUNK, chunk_body, 0)


@jax.jit
def _run(cw, xw, tin, tout, tailin, tailout):
    mesh = plsc.VectorSubcoreMesh(core_axis_name="c", subcore_axis_name="s",
                                  num_cores=NC, num_subcores=NS)

    tr = pl.kernel(
        _tr_kernel,
        out_type=(jax.ShapeDtypeStruct((VPAD, W), jnp.float32),
                  jax.ShapeDtypeStruct((VPAD, W), jnp.float32)),
        mesh=mesh,
        compiler_params=pltpu.CompilerParams(needs_layout_passes=False,
                                             use_tc_tiling_on_sc=True),
        scratch_types=[
            pltpu.VMEM((8, 8, VCH), jnp.float32),
            pltpu.VMEM((8, 8, VCH), jnp.float32),
            pltpu.VMEM((VCH, W), jnp.float32),
            pltpu.VMEM((VCH, W), jnp.float32),
            pltpu.SemaphoreType.DMA,
        ],
    )
    lin, lout = tr(tin, tout, tailin, tailout)

    gd = pl.kernel(
        _sc_kernel,
        out_type=jax.ShapeDtypeStruct((NW, NCHUNK, NSUB, SUB), jnp.float32),
        mesh=mesh,
        compiler_params=pltpu.CompilerParams(needs_layout_passes=False,
                                             use_tc_tiling_on_sc=False),
        scratch_types=[
            pltpu.VMEM((NSUB, SUB), jnp.int32),
            pltpu.VMEM((NSUB, SUB), jnp.int32),
            pltpu.VMEM((NSUB, SUB, W), jnp.float32),
            pltpu.VMEM((NSUB, SUB, W), jnp.float32),
            pltpu.VMEM((NSUB, SUB), jnp.float32),
            pltpu.SemaphoreType.DMA,
        ],
    )
    return gd(cw, xw, lin, lout)


def kernel(center_words, context_words, in_embeddings, out_embeddings):
    cw = center_words.T.reshape(NW, NCHUNK, NSUB, SUB).astype(jnp.int32)
    xw = context_words.T.reshape(NW, NCHUNK, NSUB, SUB).astype(jnp.int32)
    tailin = jnp.pad(in_embeddings.T[:, VTAIL0:], ((0, 0), (0, VCH - VTAIL)))
    tailout = jnp.pad(out_embeddings.T[:, VTAIL0:], ((0, 0), (0, VCH - VTAIL)))
    scores = _run(cw, xw, in_embeddings.T, out_embeddings.T, tailin, tailout)
    return scores.reshape(L, B).T
